# baseline (speedup 1.0000x reference)
import functools

import jax
import jax.numpy as jnp
from jax import lax
from jax.experimental import pallas as pl
from jax.experimental.pallas import tpu as pltpu
from jax.experimental.pallas import tpu_sc as plsc

_LN2 = 0.6931471805599453


def _vsqrt(v):
    i = plsc.bitcast(v, jnp.int32)
    y = plsc.bitcast(jnp.int32(0x5F3759DF) - (i >> 1), jnp.float32)
    for _ in range(3):
        y = y * (1.5 - 0.5 * v * y * y)
    return v * y


def _vln(r):
    i = plsc.bitcast(r, jnp.int32)
    ex = ((i >> 23) & 0xFF) - 127
    f = plsc.bitcast((i & 0x007FFFFF) | 0x3F800000, jnp.float32)
    y = ex.astype(jnp.float32) * _LN2 + (f - 1.0)
    for _ in range(3):
        y = y - 1.0 + r * jnp.exp(-y)
    return y


def kernel(z_all, pre_knn_indices, cell_indices):
    n_cells, dim = z_all.shape
    batch = cell_indices.shape[0]
    k = pre_knn_indices.shape[1]
    kp = 32

    info = plsc.get_sparse_core_info()
    nw = info.num_cores * info.num_subcores
    bpw = batch // nw
    ch = 4
    nchunk = bpw // ch
    nidx = ch * kp

    cell32 = cell_indices.astype(jnp.int32)
    knn_pad = jnp.pad(pre_knn_indices.astype(jnp.int32), ((0, 0), (0, kp - k)))
    neg = jax.random.randint(jax.random.key(1234), (batch, k), 0, n_cells,
                             dtype=jnp.int32)
    neg_flat = jnp.pad(neg, ((0, 0), (0, kp - k))).reshape(batch * kp)

    mesh = plsc.VectorSubcoreMesh(core_axis_name="c", subcore_axis_name="s")

    @functools.partial(
        pl.kernel,
        out_type=jax.ShapeDtypeStruct((nw, 16), jnp.float32),
        mesh=mesh,
        compiler_params=pltpu.CompilerParams(needs_layout_passes=False,
                                             use_tc_tiling_on_sc=False),
        scratch_types=[
            pltpu.VMEM((bpw,), jnp.int32),
            pltpu.VMEM((bpw, dim), jnp.float32),
            pltpu.VMEM((bpw, kp), jnp.int32),
            pltpu.VMEM((bpw * kp,), jnp.int32),
            pltpu.VMEM((bpw * kp,), jnp.int32),
            pltpu.VMEM((nidx, dim), jnp.float32),
            pltpu.VMEM((nidx, dim), jnp.float32),
            pltpu.VMEM((16,), jnp.float32),
            pltpu.SemaphoreType.DMA,
        ],
    )
    def sc_kernel(z_hbm, knn_hbm, neg_hbm, cell_hbm, out_hbm,
                  cidx, zi, nnp, nnf, ngf, posb, negb, accv, sem):
        wid = lax.axis_index("s") * info.num_cores + lax.axis_index("c")
        base = pl.multiple_of(wid * bpw, bpw)
        fbase = pl.multiple_of(wid * (bpw * kp), bpw * kp)
        pltpu.sync_copy(cell_hbm.at[pl.ds(base, bpw)], cidx)
        pltpu.async_copy(z_hbm.at[cidx], zi, sem).wait()
        pltpu.async_copy(knn_hbm.at[cidx], nnp, sem).wait()
        pltpu.sync_copy(neg_hbm.at[pl.ds(fbase, bpw * kp)], ngf)
        for e in range(bpw):
            for h in range(kp // 16):
                nnf[pl.ds(e * kp + h * 16, 16)] = nnp[e, pl.ds(h * 16, 16)]

        lanes = lax.iota(jnp.int32, 16)
        zero = jnp.zeros((16,), jnp.float32)

        def chunk_body(c, acc):
            i0 = pl.multiple_of(c * nidx, nidx)
            cp_p = pltpu.make_async_copy(
                z_hbm.at[nnf.at[pl.ds(i0, nidx)]], posb, sem)
            cp_n = pltpu.make_async_copy(
                z_hbm.at[ngf.at[pl.ds(i0, nidx)]], negb, sem)
            cp_p.start()
            cp_n.start()
            cp_p.wait()
            cp_n.wait()
            for ee in range(ch):
                e = c * ch + ee
                r0 = lanes + (ee * kp)
                r1 = r0 + 16

                def dbody(qq, accs):
                    a0, a1, a2, a3 = accs
                    zq = zi[e, pl.ds(qq * 16, 16)]
                    for r in range(16):
                        zv = zq[r]
                        cols = jnp.full((16,), qq * 16 + r, jnp.int32)
                        t0 = plsc.load_gather(posb, [r0, cols]) - zv
                        t1 = plsc.load_gather(posb, [r1, cols]) - zv
                        t2 = plsc.load_gather(negb, [r0, cols]) - zv
                        t3 = plsc.load_gather(negb, [r1, cols]) - zv
                        a0 = a0 + t0 * t0
                        a1 = a1 + t1 * t1
                        a2 = a2 + t2 * t2
                        a3 = a3 + t3 * t3
                    return (a0, a1, a2, a3)

                p0, p1, q0, q1 = lax.fori_loop(
                    0, dim // 16, dbody, (zero, zero, zero, zero))
                p1 = jnp.where(lanes < k - 16, p1, 1e30)
                q1 = jnp.where(lanes < k - 16, q1, 1e30)
                d0 = _vsqrt(p0)
                d1 = _vsqrt(p1)
                d2 = _vsqrt(q0)
                d3 = _vsqrt(q1)
                m = jnp.min(jnp.minimum(jnp.minimum(d0, d1),
                                        jnp.minimum(d2, d3)))
                mv = jnp.full((16,), m, jnp.float32)
                e0v = jnp.exp(mv - d0)
                e1v = jnp.exp(mv - d1)
                e2v = jnp.exp(mv - d2)
                e3v = jnp.exp(mv - d3)
                sp = jnp.full((16,), jnp.sum(e0v + e1v), jnp.float32)
                st = sp + jnp.full((16,), jnp.sum(e2v + e3v), jnp.float32)
                ratio = st / (sp + 1e-8 * st)
                acc = acc + _vln(ratio)
            return acc

        acc = lax.fori_loop(0, nchunk, chunk_body,
                            jnp.zeros((16,), jnp.float32))
        accv[...] = acc
        pltpu.sync_copy(accv, out_hbm.at[wid])

    partial = sc_kernel(z_all, knn_pad, neg_flat, cell32)
    return jnp.sum(partial) / (16.0 * batch)


# speedup vs baseline: 1.1319x; 1.1319x over previous
import functools

import jax
import jax.numpy as jnp
from jax import lax
from jax.experimental import pallas as pl
from jax.experimental.pallas import tpu as pltpu
from jax.experimental.pallas import tpu_sc as plsc

_LN2 = 0.6931471805599453


def _vsqrt(v):
    i = plsc.bitcast(v, jnp.int32)
    y = plsc.bitcast(jnp.int32(0x5F3759DF) - (i >> 1), jnp.float32)
    for _ in range(3):
        y = y * (1.5 - 0.5 * v * y * y)
    return v * y


def _vln(r):
    i = plsc.bitcast(r, jnp.int32)
    ex = ((i >> 23) & 0xFF) - 127
    f = plsc.bitcast((i & 0x007FFFFF) | 0x3F800000, jnp.float32)
    y = ex.astype(jnp.float32) * _LN2 + (f - 1.0)
    for _ in range(3):
        y = y - 1.0 + r * jnp.exp(-y)
    return y


def kernel(z_all, pre_knn_indices, cell_indices):
    n_cells, dim = z_all.shape
    batch = cell_indices.shape[0]
    k = pre_knn_indices.shape[1]
    kp = 32

    info = plsc.get_sparse_core_info()
    nw = info.num_cores * info.num_subcores
    bpw = batch // nw
    ch = 4
    nchunk = bpw // ch
    nidx = ch * kp

    cell32 = cell_indices.astype(jnp.int32)
    knn32 = pre_knn_indices.astype(jnp.int32)
    knn_pad = jnp.concatenate([knn32, knn32[:, :kp - k]], axis=1)
    neg = jax.random.randint(jax.random.key(1234), (batch, k), 0, n_cells,
                             dtype=jnp.int32)
    neg_flat = jnp.concatenate([neg, neg[:, :kp - k]],
                               axis=1).reshape(batch * kp)

    mesh = plsc.VectorSubcoreMesh(core_axis_name="c", subcore_axis_name="s")

    @functools.partial(
        pl.kernel,
        out_type=jax.ShapeDtypeStruct((nw, 16), jnp.float32),
        mesh=mesh,
        compiler_params=pltpu.CompilerParams(needs_layout_passes=False,
                                             use_tc_tiling_on_sc=False),
        scratch_types=[
            pltpu.VMEM((bpw,), jnp.int32),
            pltpu.VMEM((bpw, dim), jnp.float32),
            pltpu.VMEM((bpw, kp), jnp.int32),
            pltpu.VMEM((bpw * kp,), jnp.int32),
            pltpu.VMEM((bpw * kp,), jnp.int32),
            pltpu.VMEM((nidx, dim), jnp.float32),
            pltpu.VMEM((nidx, dim), jnp.float32),
            pltpu.VMEM((16,), jnp.float32),
            pltpu.SemaphoreType.DMA,
        ],
    )
    def sc_kernel(z_hbm, knn_hbm, neg_hbm, cell_hbm, out_hbm,
                  cidx, zi, nnp, nnf, ngf, posb, negb, accv, sem):
        wid = lax.axis_index("s") * info.num_cores + lax.axis_index("c")
        base = pl.multiple_of(wid * bpw, bpw)
        fbase = pl.multiple_of(wid * (bpw * kp), bpw * kp)
        pltpu.sync_copy(cell_hbm.at[pl.ds(base, bpw)], cidx)
        pltpu.async_copy(z_hbm.at[cidx], zi, sem).wait()
        pltpu.async_copy(knn_hbm.at[cidx], nnp, sem).wait()
        pltpu.sync_copy(neg_hbm.at[pl.ds(fbase, bpw * kp)], ngf)
        for e in range(bpw):
            for h in range(kp // 16):
                nnf[pl.ds(e * kp + h * 16, 16)] = nnp[e, pl.ds(h * 16, 16)]

        lanes = lax.iota(jnp.int32, 16)
        zero = jnp.zeros((16,), jnp.float32)

        def chunk_body(c, acc):
            i0 = pl.multiple_of(c * nidx, nidx)
            cp_p = pltpu.make_async_copy(
                z_hbm.at[nnf.at[pl.ds(i0, nidx)]], posb, sem)
            cp_n = pltpu.make_async_copy(
                z_hbm.at[ngf.at[pl.ds(i0, nidx)]], negb, sem)
            cp_p.start()
            cp_n.start()
            cp_p.wait()
            cp_n.wait()
            for ee in range(ch):
                e = c * ch + ee
                r0 = lanes + (ee * kp)
                r1 = r0 + 16

                def dbody(qq, accs):
                    a0, a1, a2, a3 = accs
                    zq = zi[e, pl.ds(qq * 16, 16)]
                    for r in range(16):
                        zv = zq[r]
                        cols = jnp.full((16,), qq * 16 + r, jnp.int32)
                        t0 = plsc.load_gather(posb, [r0, cols]) - zv
                        t1 = plsc.load_gather(posb, [r1, cols]) - zv
                        t2 = plsc.load_gather(negb, [r0, cols]) - zv
                        t3 = plsc.load_gather(negb, [r1, cols]) - zv
                        a0 = a0 + t0 * t0
                        a1 = a1 + t1 * t1
                        a2 = a2 + t2 * t2
                        a3 = a3 + t3 * t3
                    return (a0, a1, a2, a3)

                p0, p1, q0, q1 = lax.fori_loop(
                    0, dim // 16, dbody, (zero, zero, zero, zero))
                p1 = jnp.where(lanes < k - 16, p1, 1e30)
                q1 = jnp.where(lanes < k - 16, q1, 1e30)
                d0 = _vsqrt(p0)
                d1 = _vsqrt(p1)
                d2 = _vsqrt(q0)
                d3 = _vsqrt(q1)
                m = jnp.min(jnp.minimum(jnp.minimum(d0, d1),
                                        jnp.minimum(d2, d3)))
                mv = jnp.full((16,), m, jnp.float32)
                e0v = jnp.exp(mv - d0)
                e1v = jnp.exp(mv - d1)
                e2v = jnp.exp(mv - d2)
                e3v = jnp.exp(mv - d3)
                sp = jnp.full((16,), jnp.sum(e0v + e1v), jnp.float32)
                st = sp + jnp.full((16,), jnp.sum(e2v + e3v), jnp.float32)
                ratio = st / (sp + 1e-8 * st)
                acc = acc + _vln(ratio)
            return acc

        acc = lax.fori_loop(0, nchunk, chunk_body,
                            jnp.zeros((16,), jnp.float32))
        accv[...] = acc
        pltpu.sync_copy(accv, out_hbm.at[wid])

    partial = sc_kernel(z_all, knn_pad, neg_flat, cell32)
    return jnp.sum(partial) / (16.0 * batch)


# speedup vs baseline: 1.1923x; 1.0533x over previous
import functools

import jax
import jax.numpy as jnp
from jax import lax
from jax.experimental import pallas as pl
from jax.experimental.pallas import tpu as pltpu
from jax.experimental.pallas import tpu_sc as plsc

_LN2 = 0.6931471805599453


def _vsqrt(v):
    i = plsc.bitcast(v, jnp.int32)
    y = plsc.bitcast(jnp.int32(0x5F3759DF) - (i >> 1), jnp.float32)
    for _ in range(3):
        y = y * (1.5 - 0.5 * v * y * y)
    return v * y


def _vln(r):
    i = plsc.bitcast(r, jnp.int32)
    ex = ((i >> 23) & 0xFF) - 127
    f = plsc.bitcast((i & 0x007FFFFF) | 0x3F800000, jnp.float32)
    y = ex.astype(jnp.float32) * _LN2 + (f - 1.0)
    for _ in range(3):
        y = y - 1.0 + r * jnp.exp(-y)
    return y


def kernel(z_all, pre_knn_indices, cell_indices):
    n_cells, dim = z_all.shape
    batch = cell_indices.shape[0]
    k = pre_knn_indices.shape[1]
    kp = 32

    info = plsc.get_sparse_core_info()
    nw = info.num_cores * info.num_subcores
    bpw = batch // nw
    ch = 4
    nchunk = bpw // ch
    nidx = ch * kp

    cell32 = cell_indices.astype(jnp.int32)
    knn32 = pre_knn_indices.astype(jnp.int32)
    knn_pad = jnp.concatenate([knn32, knn32[:, :kp - k]], axis=1)
    neg = jax.random.randint(jax.random.key(1234), (batch, k), 0, n_cells,
                             dtype=jnp.int32)
    neg_flat = jnp.concatenate([neg, neg[:, :kp - k]],
                               axis=1).reshape(batch * kp)

    mesh = plsc.VectorSubcoreMesh(core_axis_name="c", subcore_axis_name="s")

    @functools.partial(
        pl.kernel,
        out_type=jax.ShapeDtypeStruct((nw, 16), jnp.float32),
        mesh=mesh,
        compiler_params=pltpu.CompilerParams(needs_layout_passes=False,
                                             use_tc_tiling_on_sc=False),
        scratch_types=[
            pltpu.VMEM((bpw,), jnp.int32),
            pltpu.VMEM((bpw, dim), jnp.float32),
            pltpu.VMEM((bpw, kp), jnp.int32),
            pltpu.VMEM((bpw * kp,), jnp.int32),
            pltpu.VMEM((bpw * kp,), jnp.int32),
            pltpu.VMEM((4, nidx, dim), jnp.float32),
            pltpu.VMEM((4, nidx, dim), jnp.float32),
            pltpu.VMEM((16,), jnp.float32),
            pltpu.SemaphoreType.DMA,
            pltpu.SemaphoreType.DMA,
            pltpu.SemaphoreType.DMA,
            pltpu.SemaphoreType.DMA,
            pltpu.SemaphoreType.DMA,
            pltpu.SemaphoreType.DMA,
            pltpu.SemaphoreType.DMA,
            pltpu.SemaphoreType.DMA,
            pltpu.SemaphoreType.DMA,
        ],
    )
    def sc_kernel(z_hbm, knn_hbm, neg_hbm, cell_hbm, out_hbm,
                  cidx, zi, nnp, nnf, ngf, posb, negb, accv, sem,
                  ps0, ps1, ps2, ps3, ns0, ns1, ns2, ns3):
        wid = lax.axis_index("s") * info.num_cores + lax.axis_index("c")
        base = pl.multiple_of(wid * bpw, bpw)
        fbase = pl.multiple_of(wid * (bpw * kp), bpw * kp)
        pltpu.sync_copy(cell_hbm.at[pl.ds(base, bpw)], cidx)
        pltpu.async_copy(z_hbm.at[cidx], zi, sem).wait()
        pltpu.async_copy(knn_hbm.at[cidx], nnp, sem).wait()
        pltpu.sync_copy(neg_hbm.at[pl.ds(fbase, bpw * kp)], ngf)
        for e in range(bpw):
            for h in range(kp // 16):
                nnf[pl.ds(e * kp + h * 16, 16)] = nnp[e, pl.ds(h * 16, 16)]

        lanes = lax.iota(jnp.int32, 16)
        zero = jnp.zeros((16,), jnp.float32)
        psems = [ps0, ps1, ps2, ps3]
        nsems = [ns0, ns1, ns2, ns3]
        nb = 4

        def cp_pos(c, b):
            i0 = pl.multiple_of(c * nidx, nidx)
            return pltpu.make_async_copy(
                z_hbm.at[nnf.at[pl.ds(i0, nidx)]], posb.at[b], psems[b])

        def cp_neg(c, b):
            i0 = pl.multiple_of(c * nidx, nidx)
            return pltpu.make_async_copy(
                z_hbm.at[ngf.at[pl.ds(i0, nidx)]], negb.at[b], nsems[b])

        for b in range(nb):
            cp_pos(b, b).start()
            cp_neg(b, b).start()

        def compute_chunk(c, b, acc):
            bv = jnp.full((16,), b, jnp.int32)
            for ee in range(ch):
                e = c * ch + ee
                r0 = lanes + (ee * kp)
                r1 = r0 + 16

                def dbody(qq, accs):
                    a0, a1, a2, a3 = accs
                    zq = zi[e, pl.ds(qq * 16, 16)]
                    for r in range(16):
                        zv = zq[r]
                        cols = jnp.full((16,), qq * 16 + r, jnp.int32)
                        t0 = plsc.load_gather(posb, [bv, r0, cols]) - zv
                        t1 = plsc.load_gather(posb, [bv, r1, cols]) - zv
                        t2 = plsc.load_gather(negb, [bv, r0, cols]) - zv
                        t3 = plsc.load_gather(negb, [bv, r1, cols]) - zv
                        a0 = a0 + t0 * t0
                        a1 = a1 + t1 * t1
                        a2 = a2 + t2 * t2
                        a3 = a3 + t3 * t3
                    return (a0, a1, a2, a3)

                p0, p1, q0, q1 = lax.fori_loop(
                    0, dim // 16, dbody, (zero, zero, zero, zero))
                p1 = jnp.where(lanes < k - 16, p1, 1e30)
                q1 = jnp.where(lanes < k - 16, q1, 1e30)
                d0 = _vsqrt(p0)
                d1 = _vsqrt(p1)
                d2 = _vsqrt(q0)
                d3 = _vsqrt(q1)
                m = jnp.min(jnp.minimum(jnp.minimum(d0, d1),
                                        jnp.minimum(d2, d3)))
                mv = jnp.full((16,), m, jnp.float32)
                e0v = jnp.exp(mv - d0)
                e1v = jnp.exp(mv - d1)
                e2v = jnp.exp(mv - d2)
                e3v = jnp.exp(mv - d3)
                sp = jnp.full((16,), jnp.sum(e0v + e1v), jnp.float32)
                st = sp + jnp.full((16,), jnp.sum(e2v + e3v), jnp.float32)
                ratio = st / (sp + 1e-8 * st)
                acc = acc + _vln(ratio)
            return acc

        def group_body(g, acc):
            c0 = g * nb
            for b in range(nb):
                c = c0 + b
                cp_pos(c, b).wait()
                cp_neg(c, b).wait()
                acc = compute_chunk(c, b, acc)
                nxt = c + nb

                @pl.when(nxt < nchunk)
                def _():
                    cp_pos(nxt, b).start()
                    cp_neg(nxt, b).start()
            return acc

        acc = lax.fori_loop(0, nchunk // nb, group_body,
                            jnp.zeros((16,), jnp.float32))
        accv[...] = acc
        pltpu.sync_copy(accv, out_hbm.at[wid])

    partial = sc_kernel(z_all, knn_pad, neg_flat, cell32)
    return jnp.sum(partial) / (16.0 * batch)
